# baseline (speedup 1.0000x reference)
import jax
import jax.numpy as jnp
from jax.experimental import pallas as pl

N = 50000
D_FEAT = 128
DEG = 32
D_EDGE = 16
OUT_HALF = 64
MID = 96
BLK = 512


def _tc_body(nf, mb, w1at, b1a, w1bt, b1b, w2at, b2a, w2bt, b2b, out):
    x = nf[...]
    h = jnp.maximum(
        jnp.dot(x, w1at[...], preferred_element_type=jnp.float32) + b1a[...], 0.0)
    r1 = jnp.tanh(jnp.dot(h, w1bt[...], preferred_element_type=jnp.float32) + b1b[...])

    r_idx = jax.lax.broadcasted_iota(jnp.int32, (DEG * D_EDGE, D_EDGE), 0)
    d_idx = jax.lax.broadcasted_iota(jnp.int32, (DEG * D_EDGE, D_EDGE), 1)
    sel = (r_idx % D_EDGE == d_idx).astype(jnp.float32)
    agg = jnp.dot(mb[...], sel, preferred_element_type=jnp.float32)

    h2 = jnp.maximum(
        jnp.dot(agg, w2at[...], preferred_element_type=jnp.float32) + b2a[...], 0.0)
    r2 = jnp.tanh(jnp.dot(h2, w2bt[...], preferred_element_type=jnp.float32) + b2b[...])

    res = jnp.concatenate([r1, r2], axis=1)
    inv = jax.lax.rsqrt(jnp.sum(res * res, axis=1, keepdims=True))
    out[...] = res * inv


def kernel(node_features, mailbox, W1a, b1a, W1b, b1b, W2a, b2a, W2b, b2b):
    n = node_features.shape[0]
    mb = mailbox.reshape(n, DEG * D_EDGE)
    grid = (pl.cdiv(n, BLK),)

    def full(shape):
        return pl.BlockSpec(shape, lambda i: (0,) * len(shape))

    out = pl.pallas_call(
        _tc_body,
        grid=grid,
        in_specs=[
            pl.BlockSpec((BLK, D_FEAT), lambda i: (i, 0)),
            pl.BlockSpec((BLK, DEG * D_EDGE), lambda i: (i, 0)),
            full((D_FEAT, MID)),
            full((1, MID)),
            full((MID, OUT_HALF)),
            full((1, OUT_HALF)),
            full((D_EDGE, OUT_HALF)),
            full((1, OUT_HALF)),
            full((OUT_HALF, OUT_HALF)),
            full((1, OUT_HALF)),
        ],
        out_specs=pl.BlockSpec((BLK, D_FEAT), lambda i: (i, 0)),
        out_shape=jax.ShapeDtypeStruct((n, D_FEAT), jnp.float32),
    )(node_features, mb,
      W1a.T, b1a.reshape(1, MID),
      W1b.T, b1b.reshape(1, OUT_HALF),
      W2a.T, b2a.reshape(1, OUT_HALF),
      W2b.T, b2b.reshape(1, OUT_HALF))
    return out
